# baseline (speedup 1.0000x reference)
import jax
import jax.numpy as jnp
import numpy as np
from jax.experimental import pallas as pl
from jax.experimental.pallas import tpu as pltpu

E = 8
D = 768
H = 12
DH = D // H
DR = D // 2
S = 2048
NA = 2 * S
BA = 64
NB = NA // BA
P_MAX = NB + E - 1
BP = 512
BT = 256


def _bdot(a, b, dims):
    return jax.lax.dot_general(
        a.astype(jnp.bfloat16), b.astype(jnp.bfloat16),
        dimension_numbers=(dims, ((), ())),
        preferred_element_type=jnp.float32)


def _shift_down_lanes(a, k):
    return jnp.concatenate([jnp.zeros((a.shape[0], k), a.dtype), a[:, :-k]],
                           axis=1)


def _shift_down_subl(a, k):
    return jnp.concatenate([jnp.zeros((k, a.shape[1]), a.dtype), a[:-k, :]],
                           axis=0)


def _router_body(xb_ref, w1_ref, b1_ref, g_ref, be_ref, w2_ref, b2_ref,
                 p1c_ref, p2c_ref, g1c_ref, g2c_ref, off_ref,
                 blk_ref, exp_ref, act_ref, ini_ref):
    h = _bdot(xb_ref[...], w1_ref[...], ((1,), (1,))) + b1_ref[...]
    mu = jnp.mean(h, axis=-1, keepdims=True)
    var = jnp.mean((h - mu) ** 2, axis=-1, keepdims=True)
    h = (h - mu) / jnp.sqrt(var + 1e-5) * g_ref[...] + be_ref[...]
    h = jax.nn.relu(h)
    logits = _bdot(h, w2_ref[...], ((1,), (1,))) + b2_ref[...]
    probs = jax.nn.softmax(logits, axis=-1)
    ei = jax.lax.broadcasted_iota(jnp.int32, probs.shape, 1)
    m1 = jnp.max(probs, axis=1, keepdims=True)
    i1 = jnp.min(jnp.where(probs == m1, ei, E), axis=1, keepdims=True)
    probs2 = jnp.where(ei == i1, -jnp.inf, probs)
    m2 = jnp.max(probs2, axis=1, keepdims=True)
    i2 = jnp.min(jnp.where(probs2 == m2, ei, E), axis=1, keepdims=True)
    denom = m1 + m2
    g1c_ref[...] = m1 / denom
    g2c_ref[...] = m2 / denom

    ind = ((ei == i1) | (ei == i2)).astype(jnp.float32)
    counts = jnp.sum(ind, axis=0, keepdims=True)
    incl = counts
    for k in (1, 2, 4):
        incl = incl + _shift_down_lanes(incl, k)
    off = jnp.concatenate([jnp.zeros((1, 1), jnp.float32), incl], axis=1)
    off_ref[...] = off.astype(jnp.int32)

    itc = jax.lax.broadcasted_iota(jnp.int32, (S, S), 0)
    itr = jax.lax.broadcasted_iota(jnp.int32, (S, S), 1)
    ltri = (itr < itc).astype(jnp.bfloat16)
    rank = jax.lax.dot_general(
        ltri, ind.astype(jnp.bfloat16), (((1,), (0,)), ((), ())),
        preferred_element_type=jnp.float32)
    pos = off[:, :E] + rank
    p1c_ref[...] = jnp.sum(jnp.where(ei == i1, pos, 0.0), axis=1,
                           keepdims=True).astype(jnp.int32)
    p2c_ref[...] = jnp.sum(jnp.where(ei == i2, pos, 0.0), axis=1,
                           keepdims=True).astype(jnp.int32)

    edge_lo = jax.lax.broadcasted_iota(jnp.int32, (NB, 1), 0).astype(
        jnp.float32) * BA
    edge_hi = edge_lo + (BA - 1)
    e_lo = jnp.sum((edge_lo >= off).astype(jnp.float32), axis=1,
                   keepdims=True) - 1.0
    e_hi = jnp.sum((edge_hi >= off).astype(jnp.float32), axis=1,
                   keepdims=True) - 1.0
    n = e_hi - e_lo + 1.0
    cincl = n
    k = 1
    while k < NB:
        cincl = cincl + _shift_down_subl(cincl, k)
        k *= 2
    cexcl = cincl - n
    total = cincl[NB - 1, 0]
    prow = jax.lax.broadcasted_iota(jnp.int32, (1, P_MAX), 1).astype(
        jnp.float32)
    peff = jnp.minimum(prow, total - 1.0)
    in_bp = (peff >= cexcl) & (peff < cincl)
    bcol = jax.lax.broadcasted_iota(jnp.int32, (NB, 1), 0).astype(jnp.float32)
    blk = jnp.sum(jnp.where(in_bp, bcol, 0.0), axis=0, keepdims=True)
    ex = jnp.sum(jnp.where(in_bp, e_lo + peff - cexcl, 0.0), axis=0,
                 keepdims=True)
    act = (prow < total).astype(jnp.int32)
    blk_prev = jnp.concatenate([blk[:, :1] - 1.0, blk[:, :-1]], axis=1)
    ini = (blk != blk_prev).astype(jnp.int32)
    blk_ref[...] = blk.astype(jnp.int32)
    exp_ref[...] = ex.astype(jnp.int32)
    act_ref[...] = act
    ini_ref[...] = ini


def _kv_dispatch_body(xb_ref, p1_ref, p2_ref, wk_ref, bk_ref, wv_ref, bv_ref,
                      k_out, v_out, km_out, xs_out):
    e = pl.program_id(0)
    k = _bdot(xb_ref[...], wk_ref[0], ((1,), (1,))) + bk_ref[0]
    k_out[0] = k.astype(jnp.bfloat16)
    k2 = k * k
    dcol = jax.lax.broadcasted_iota(jnp.int32, (D, H), 0)
    hrow = jax.lax.broadcasted_iota(jnp.int32, (D, H), 1)
    hm = (dcol // DH == hrow).astype(jnp.float32)
    kn2 = _bdot(k2, hm, ((1,), (0,)))
    km_out[0] = jnp.max(kn2, axis=0, keepdims=True)
    v = _bdot(xb_ref[...], wv_ref[0], ((1,), (1,))) + bv_ref[0]
    vb = v.astype(jnp.bfloat16)
    pat = jnp.concatenate([jnp.ones((S, 1), jnp.bfloat16),
                           jnp.zeros((S, DH - 1), jnp.bfloat16)], axis=1)
    for h in range(H):
        vh = jnp.concatenate([vb[:, h * DH:(h + 1) * DH], pat], axis=1)
        v_out[0, :, 2 * h * DH:(2 * h + 2) * DH] = vh
    prow = e * BP + jax.lax.broadcasted_iota(jnp.int32, (1, BP), 1)
    eqt = ((p1_ref[...] == prow) | (p2_ref[...] == prow)).astype(jnp.bfloat16)
    xs = jax.lax.dot_general(
        eqt, xb_ref[...], (((0,), (0,)), ((), ())),
        preferred_element_type=jnp.float32)
    xs_out[...] = xs.astype(jnp.bfloat16)


def _attn_body(blk_s, exp_s, act_s, ini_s, off_s,
               xs_ref, k_ref, v_ref, km_ref, wq_ref, bq_ref, wo_ref, bo_ref,
               y_ref):
    p = pl.program_id(0)
    e = exp_s[p]
    scale = 1.0 / np.sqrt(DH)
    q = _bdot(xs_ref[...], wq_ref[0], ((1,), (1,))) + bq_ref[0]
    qs = q * scale
    qb = qs.astype(jnp.bfloat16)
    q2 = qs * qs
    km = km_ref[0]
    o_parts = []
    for h in range(H):
        sl = slice(h * DH, (h + 1) * DH)
        scores = jax.lax.dot_general(
            qb[:, sl], k_ref[0][:, sl], (((1,), (1,)), ((), ())),
            preferred_element_type=jnp.float32)
        qn2 = jnp.sum(q2[:, sl], axis=1, keepdims=True)
        b = jnp.sqrt(qn2 * km[0, h]) + 1.0
        ex = jnp.exp(scores - b)
        av = jax.lax.dot_general(
            ex.astype(jnp.bfloat16), v_ref[0][:, h * 2 * DH:(h + 1) * 2 * DH],
            (((1,), (0,)), ((), ())), preferred_element_type=jnp.float32)
        oh = av[:, :DH] / av[:, DH:DH + 1]
        o_parts.append(oh)
    o = jnp.concatenate(o_parts, axis=1)
    o = _bdot(o, wo_ref[0], ((1,), (1,))) + bo_ref[0]
    piota = blk_s[p] * BA + jax.lax.broadcasted_iota(jnp.int32, (BA, 1), 0)
    rowmask = (piota >= off_s[e]) & (piota < off_s[e + 1]) & (act_s[p] > 0)
    contrib = jnp.where(rowmask, o, 0.0)

    @pl.when(ini_s[p] == 1)
    def _():
        y_ref[...] = contrib

    @pl.when(ini_s[p] == 0)
    def _():
        y_ref[...] = y_ref[...] + contrib


def _combine_body(yb_ref, p1_ref, p2_ref, g1_ref, g2_ref, out_ref):
    prow = jax.lax.broadcasted_iota(jnp.int32, (1, NA), 1)
    eq1 = (p1_ref[...] == prow).astype(jnp.bfloat16)
    eq2 = (p2_ref[...] == prow).astype(jnp.bfloat16)
    a1 = jax.lax.dot_general(eq1, yb_ref[...], (((1,), (0,)), ((), ())),
                             preferred_element_type=jnp.float32)
    a2 = jax.lax.dot_general(eq2, yb_ref[...], (((1,), (0,)), ((), ())),
                             preferred_element_type=jnp.float32)
    out_ref[...] = g1_ref[...] * a1 + g2_ref[...] * a2


def kernel(x, Wq, bq, Wk, bk, Wv, bv, Wo, bo, rW1, rb1, ln_g, ln_b, rW2, rb2):
    x2 = x[0]
    xb = x2.astype(jnp.bfloat16)
    i32 = jnp.int32
    f32 = jnp.float32

    p1c, p2c, g1c, g2c, off, blk, ex, act, ini = pl.pallas_call(
        _router_body,
        out_shape=(
            jax.ShapeDtypeStruct((S, 1), i32),
            jax.ShapeDtypeStruct((S, 1), i32),
            jax.ShapeDtypeStruct((S, 1), f32),
            jax.ShapeDtypeStruct((S, 1), f32),
            jax.ShapeDtypeStruct((1, E + 1), i32),
            jax.ShapeDtypeStruct((1, P_MAX), i32),
            jax.ShapeDtypeStruct((1, P_MAX), i32),
            jax.ShapeDtypeStruct((1, P_MAX), i32),
            jax.ShapeDtypeStruct((1, P_MAX), i32),
        ),
    )(xb, rW1, rb1.reshape(1, DR), ln_g.reshape(1, DR),
      ln_b.reshape(1, DR), rW2, rb2.reshape(1, E))

    wk16 = Wk.astype(jnp.bfloat16)
    wv16 = Wv.astype(jnp.bfloat16)
    wq16 = Wq.astype(jnp.bfloat16)
    wo16 = Wo.astype(jnp.bfloat16)
    b3 = lambda a: a.reshape(E, 1, D)

    wspec = pl.BlockSpec((1, D, D), lambda e: (e, 0, 0))
    bspec = pl.BlockSpec((1, 1, D), lambda e: (e, 0, 0))
    kall, vall, kmax, xs = pl.pallas_call(
        _kv_dispatch_body,
        grid=(E,),
        in_specs=[pl.BlockSpec((S, D), lambda e: (0, 0)),
                  pl.BlockSpec((S, 1), lambda e: (0, 0)),
                  pl.BlockSpec((S, 1), lambda e: (0, 0)),
                  wspec, bspec, wspec, bspec],
        out_specs=(pl.BlockSpec((1, S, D), lambda e: (e, 0, 0)),
                   pl.BlockSpec((1, S, 2 * D), lambda e: (e, 0, 0)),
                   pl.BlockSpec((1, 1, H), lambda e: (e, 0, 0)),
                   pl.BlockSpec((BP, D), lambda e: (e, 0))),
        out_shape=(jax.ShapeDtypeStruct((E, S, D), jnp.bfloat16),
                   jax.ShapeDtypeStruct((E, S, 2 * D), jnp.bfloat16),
                   jax.ShapeDtypeStruct((E, 1, H), jnp.float32),
                   jax.ShapeDtypeStruct((NA, D), jnp.bfloat16)),
    )(xb, p1c, p2c, wk16, b3(bk), wv16, b3(bv))

    grid_spec = pltpu.PrefetchScalarGridSpec(
        num_scalar_prefetch=5,
        grid=(P_MAX,),
        in_specs=[
            pl.BlockSpec((BA, D), lambda p, b, e, a, i, o: (b[p], 0)),
            pl.BlockSpec((1, S, D), lambda p, b, e, a, i, o: (e[p], 0, 0)),
            pl.BlockSpec((1, S, 2 * D), lambda p, b, e, a, i, o: (e[p], 0, 0)),
            pl.BlockSpec((1, 1, H), lambda p, b, e, a, i, o: (e[p], 0, 0)),
            pl.BlockSpec((1, D, D), lambda p, b, e, a, i, o: (e[p], 0, 0)),
            pl.BlockSpec((1, 1, D), lambda p, b, e, a, i, o: (e[p], 0, 0)),
            pl.BlockSpec((1, D, D), lambda p, b, e, a, i, o: (e[p], 0, 0)),
            pl.BlockSpec((1, 1, D), lambda p, b, e, a, i, o: (e[p], 0, 0)),
        ],
        out_specs=pl.BlockSpec((BA, D), lambda p, b, e, a, i, o: (b[p], 0)),
    )
    y = pl.pallas_call(
        _attn_body,
        grid_spec=grid_spec,
        out_shape=jax.ShapeDtypeStruct((NA, D), f32),
    )(blk.reshape(P_MAX), ex.reshape(P_MAX), act.reshape(P_MAX),
      ini.reshape(P_MAX), off.reshape(E + 1),
      xs, kall, vall, kmax, wq16, b3(bq), wo16, b3(bo))

    out = pl.pallas_call(
        _combine_body,
        grid=(S // BT,),
        in_specs=[pl.BlockSpec((NA, D), lambda j: (0, 0)),
                  pl.BlockSpec((BT, 1), lambda j: (j, 0)),
                  pl.BlockSpec((BT, 1), lambda j: (j, 0)),
                  pl.BlockSpec((BT, 1), lambda j: (j, 0)),
                  pl.BlockSpec((BT, 1), lambda j: (j, 0))],
        out_specs=pl.BlockSpec((BT, D), lambda j: (j, 0)),
        out_shape=jax.ShapeDtypeStruct((S, D), f32),
    )(y.astype(jnp.bfloat16), p1c, p2c, g1c, g2c)
    return out.reshape(1, S, D)


# speedup vs baseline: 1.6473x; 1.6473x over previous
import jax
import jax.numpy as jnp
import numpy as np
from jax.experimental import pallas as pl
from jax.experimental.pallas import tpu as pltpu

E = 8
D = 768
H = 12
DH = D // H
DR = D // 2
S = 2048
NA = 2 * S
BA = 128
NB = NA // BA
P_MAX = NB + E - 1
BP = 512
BT = 256


def _bdot(a, b, dims):
    return jax.lax.dot_general(
        a.astype(jnp.bfloat16), b.astype(jnp.bfloat16),
        dimension_numbers=(dims, ((), ())),
        preferred_element_type=jnp.float32)


def _shift_down_lanes(a, k):
    return jnp.concatenate([jnp.zeros((a.shape[0], k), a.dtype), a[:, :-k]],
                           axis=1)


def _shift_down_subl(a, k):
    return jnp.concatenate([jnp.zeros((k, a.shape[1]), a.dtype), a[:-k, :]],
                           axis=0)


def _router_body(xb_ref, w1_ref, b1_ref, g_ref, be_ref, w2_ref, b2_ref,
                 p1c_ref, p2c_ref, g1c_ref, g2c_ref, off_ref,
                 blk_ref, exp_ref, act_ref, ini_ref):
    h = _bdot(xb_ref[...], w1_ref[...], ((1,), (1,))) + b1_ref[...]
    mu = jnp.mean(h, axis=-1, keepdims=True)
    var = jnp.mean((h - mu) ** 2, axis=-1, keepdims=True)
    h = (h - mu) / jnp.sqrt(var + 1e-5) * g_ref[...] + be_ref[...]
    h = jax.nn.relu(h)
    logits = _bdot(h, w2_ref[...], ((1,), (1,))) + b2_ref[...]
    probs = jax.nn.softmax(logits, axis=-1)
    ei = jax.lax.broadcasted_iota(jnp.int32, probs.shape, 1)
    m1 = jnp.max(probs, axis=1, keepdims=True)
    i1 = jnp.min(jnp.where(probs == m1, ei, E), axis=1, keepdims=True)
    probs2 = jnp.where(ei == i1, -jnp.inf, probs)
    m2 = jnp.max(probs2, axis=1, keepdims=True)
    i2 = jnp.min(jnp.where(probs2 == m2, ei, E), axis=1, keepdims=True)
    denom = m1 + m2
    g1c_ref[...] = m1 / denom
    g2c_ref[...] = m2 / denom

    ind = ((ei == i1) | (ei == i2)).astype(jnp.float32)
    counts = jnp.sum(ind, axis=0, keepdims=True)
    incl = counts
    for k in (1, 2, 4):
        incl = incl + _shift_down_lanes(incl, k)
    off = jnp.concatenate([jnp.zeros((1, 1), jnp.float32), incl], axis=1)
    off_ref[...] = off.astype(jnp.int32)

    itc = jax.lax.broadcasted_iota(jnp.int32, (S, S), 0)
    itr = jax.lax.broadcasted_iota(jnp.int32, (S, S), 1)
    ltri = (itr < itc).astype(jnp.bfloat16)
    rank = jax.lax.dot_general(
        ltri, ind.astype(jnp.bfloat16), (((1,), (0,)), ((), ())),
        preferred_element_type=jnp.float32)
    pos = off[:, :E] + rank
    p1c_ref[...] = jnp.sum(jnp.where(ei == i1, pos, 0.0), axis=1,
                           keepdims=True).astype(jnp.int32)
    p2c_ref[...] = jnp.sum(jnp.where(ei == i2, pos, 0.0), axis=1,
                           keepdims=True).astype(jnp.int32)

    edge_lo = jax.lax.broadcasted_iota(jnp.int32, (NB, 1), 0).astype(
        jnp.float32) * BA
    edge_hi = edge_lo + (BA - 1)
    e_lo = jnp.sum((edge_lo >= off).astype(jnp.float32), axis=1,
                   keepdims=True) - 1.0
    e_hi = jnp.sum((edge_hi >= off).astype(jnp.float32), axis=1,
                   keepdims=True) - 1.0
    n = e_hi - e_lo + 1.0
    cincl = n
    k = 1
    while k < NB:
        cincl = cincl + _shift_down_subl(cincl, k)
        k *= 2
    cexcl = cincl - n
    total = cincl[NB - 1, 0]
    prow = jax.lax.broadcasted_iota(jnp.int32, (1, P_MAX), 1).astype(
        jnp.float32)
    peff = jnp.minimum(prow, total - 1.0)
    in_bp = (peff >= cexcl) & (peff < cincl)
    bcol = jax.lax.broadcasted_iota(jnp.int32, (NB, 1), 0).astype(jnp.float32)
    blk = jnp.sum(jnp.where(in_bp, bcol, 0.0), axis=0, keepdims=True)
    ex = jnp.sum(jnp.where(in_bp, e_lo + peff - cexcl, 0.0), axis=0,
                 keepdims=True)
    act = (prow < total).astype(jnp.int32)
    blk_prev = jnp.concatenate([blk[:, :1] - 1.0, blk[:, :-1]], axis=1)
    ini = (blk != blk_prev).astype(jnp.int32)
    blk_ref[...] = blk.astype(jnp.int32)
    exp_ref[...] = ex.astype(jnp.int32)
    act_ref[...] = act
    ini_ref[...] = ini


def _kv_dispatch_body(xb_ref, p1_ref, p2_ref, wk_ref, bk_ref, wv_ref, bv_ref,
                      k_out, v_out, km_out, xs_out):
    e = pl.program_id(0)
    k = _bdot(xb_ref[...], wk_ref[0], ((1,), (1,))) + bk_ref[0]
    k_out[0] = k.astype(jnp.bfloat16)
    k2 = k * k
    dcol = jax.lax.broadcasted_iota(jnp.int32, (D, H), 0)
    hrow = jax.lax.broadcasted_iota(jnp.int32, (D, H), 1)
    hm = (dcol // DH == hrow).astype(jnp.float32)
    kn2 = _bdot(k2, hm, ((1,), (0,)))
    km_out[0] = jnp.max(kn2, axis=0, keepdims=True)
    v = _bdot(xb_ref[...], wv_ref[0], ((1,), (1,))) + bv_ref[0]
    vb = v.astype(jnp.bfloat16)
    pat = jnp.concatenate([jnp.ones((S, 1), jnp.bfloat16),
                           jnp.zeros((S, DH - 1), jnp.bfloat16)], axis=1)
    for h in range(H):
        vh = jnp.concatenate([vb[:, h * DH:(h + 1) * DH], pat], axis=1)
        v_out[0, :, 2 * h * DH:(2 * h + 2) * DH] = vh
    prow = e * BP + jax.lax.broadcasted_iota(jnp.int32, (1, BP), 1)
    eqt = ((p1_ref[...] == prow) | (p2_ref[...] == prow)).astype(jnp.bfloat16)
    xs = jax.lax.dot_general(
        eqt, xb_ref[...], (((0,), (0,)), ((), ())),
        preferred_element_type=jnp.float32)
    xs_out[...] = xs.astype(jnp.bfloat16)


def _attn_body(blk_s, exp_s, act_s, ini_s, off_s,
               xs_ref, k_ref, v_ref, km_ref, wq_ref, bq_ref, wo_ref, bo_ref,
               y_ref):
    p = pl.program_id(0)
    e = exp_s[p]
    scale = 1.0 / np.sqrt(DH)
    q = _bdot(xs_ref[...], wq_ref[0], ((1,), (1,))) + bq_ref[0]
    qs = q * scale
    qb = qs.astype(jnp.bfloat16)
    q2 = qs * qs
    km = km_ref[0]
    o_parts = []
    for h in range(H):
        sl = slice(h * DH, (h + 1) * DH)
        scores = jax.lax.dot_general(
            qb[:, sl], k_ref[0][:, sl], (((1,), (1,)), ((), ())),
            preferred_element_type=jnp.float32)
        qn2 = jnp.sum(q2[:, sl], axis=1, keepdims=True)
        b = jnp.sqrt(qn2 * km[0, h]) + 1.0
        ex = jnp.exp(scores - b)
        av = jax.lax.dot_general(
            ex.astype(jnp.bfloat16), v_ref[0][:, h * 2 * DH:(h + 1) * 2 * DH],
            (((1,), (0,)), ((), ())), preferred_element_type=jnp.float32)
        oh = av[:, :DH] / av[:, DH:DH + 1]
        o_parts.append(oh)
    o = jnp.concatenate(o_parts, axis=1)
    o = _bdot(o, wo_ref[0], ((1,), (1,))) + bo_ref[0]
    piota = blk_s[p] * BA + jax.lax.broadcasted_iota(jnp.int32, (BA, 1), 0)
    rowmask = (piota >= off_s[e]) & (piota < off_s[e + 1]) & (act_s[p] > 0)
    contrib = jnp.where(rowmask, o, 0.0)

    @pl.when(ini_s[p] == 1)
    def _():
        y_ref[...] = contrib

    @pl.when(ini_s[p] == 0)
    def _():
        y_ref[...] = y_ref[...] + contrib


def _combine_body(yb_ref, p1_ref, p2_ref, g1_ref, g2_ref, out_ref):
    prow = jax.lax.broadcasted_iota(jnp.int32, (1, NA), 1)
    eq1 = (p1_ref[...] == prow).astype(jnp.bfloat16)
    eq2 = (p2_ref[...] == prow).astype(jnp.bfloat16)
    a1 = jax.lax.dot_general(eq1, yb_ref[...], (((1,), (0,)), ((), ())),
                             preferred_element_type=jnp.float32)
    a2 = jax.lax.dot_general(eq2, yb_ref[...], (((1,), (0,)), ((), ())),
                             preferred_element_type=jnp.float32)
    out_ref[...] = g1_ref[...] * a1 + g2_ref[...] * a2


def kernel(x, Wq, bq, Wk, bk, Wv, bv, Wo, bo, rW1, rb1, ln_g, ln_b, rW2, rb2):
    x2 = x[0]
    xb = x2.astype(jnp.bfloat16)
    i32 = jnp.int32
    f32 = jnp.float32

    p1c, p2c, g1c, g2c, off, blk, ex, act, ini = pl.pallas_call(
        _router_body,
        out_shape=(
            jax.ShapeDtypeStruct((S, 1), i32),
            jax.ShapeDtypeStruct((S, 1), i32),
            jax.ShapeDtypeStruct((S, 1), f32),
            jax.ShapeDtypeStruct((S, 1), f32),
            jax.ShapeDtypeStruct((1, E + 1), i32),
            jax.ShapeDtypeStruct((1, P_MAX), i32),
            jax.ShapeDtypeStruct((1, P_MAX), i32),
            jax.ShapeDtypeStruct((1, P_MAX), i32),
            jax.ShapeDtypeStruct((1, P_MAX), i32),
        ),
    )(xb, rW1, rb1.reshape(1, DR), ln_g.reshape(1, DR),
      ln_b.reshape(1, DR), rW2, rb2.reshape(1, E))

    b3 = lambda a: a.reshape(E, 1, D)

    wspec = pl.BlockSpec((1, D, D), lambda e: (e, 0, 0))
    bspec = pl.BlockSpec((1, 1, D), lambda e: (e, 0, 0))
    kall, vall, kmax, xs = pl.pallas_call(
        _kv_dispatch_body,
        grid=(E,),
        in_specs=[pl.BlockSpec((S, D), lambda e: (0, 0)),
                  pl.BlockSpec((S, 1), lambda e: (0, 0)),
                  pl.BlockSpec((S, 1), lambda e: (0, 0)),
                  wspec, bspec, wspec, bspec],
        out_specs=(pl.BlockSpec((1, S, D), lambda e: (e, 0, 0)),
                   pl.BlockSpec((1, S, 2 * D), lambda e: (e, 0, 0)),
                   pl.BlockSpec((1, 1, H), lambda e: (e, 0, 0)),
                   pl.BlockSpec((BP, D), lambda e: (e, 0))),
        out_shape=(jax.ShapeDtypeStruct((E, S, D), jnp.bfloat16),
                   jax.ShapeDtypeStruct((E, S, 2 * D), jnp.bfloat16),
                   jax.ShapeDtypeStruct((E, 1, H), jnp.float32),
                   jax.ShapeDtypeStruct((NA, D), jnp.bfloat16)),
    )(xb, p1c, p2c, Wk, b3(bk), Wv, b3(bv))

    grid_spec = pltpu.PrefetchScalarGridSpec(
        num_scalar_prefetch=5,
        grid=(P_MAX,),
        in_specs=[
            pl.BlockSpec((BA, D), lambda p, b, e, a, i, o: (b[p], 0)),
            pl.BlockSpec((1, S, D), lambda p, b, e, a, i, o: (e[p], 0, 0)),
            pl.BlockSpec((1, S, 2 * D), lambda p, b, e, a, i, o: (e[p], 0, 0)),
            pl.BlockSpec((1, 1, H), lambda p, b, e, a, i, o: (e[p], 0, 0)),
            pl.BlockSpec((1, D, D), lambda p, b, e, a, i, o: (e[p], 0, 0)),
            pl.BlockSpec((1, 1, D), lambda p, b, e, a, i, o: (e[p], 0, 0)),
            pl.BlockSpec((1, D, D), lambda p, b, e, a, i, o: (e[p], 0, 0)),
            pl.BlockSpec((1, 1, D), lambda p, b, e, a, i, o: (e[p], 0, 0)),
        ],
        out_specs=pl.BlockSpec((BA, D), lambda p, b, e, a, i, o: (b[p], 0)),
    )
    y = pl.pallas_call(
        _attn_body,
        grid_spec=grid_spec,
        out_shape=jax.ShapeDtypeStruct((NA, D), f32),
    )(blk.reshape(P_MAX), ex.reshape(P_MAX), act.reshape(P_MAX),
      ini.reshape(P_MAX), off.reshape(E + 1),
      xs, kall, vall, kmax, Wq, b3(bq), Wo, b3(bo))

    out = pl.pallas_call(
        _combine_body,
        grid=(S // BT,),
        in_specs=[pl.BlockSpec((NA, D), lambda j: (0, 0)),
                  pl.BlockSpec((BT, 1), lambda j: (j, 0)),
                  pl.BlockSpec((BT, 1), lambda j: (j, 0)),
                  pl.BlockSpec((BT, 1), lambda j: (j, 0)),
                  pl.BlockSpec((BT, 1), lambda j: (j, 0))],
        out_specs=pl.BlockSpec((BT, D), lambda j: (j, 0)),
        out_shape=jax.ShapeDtypeStruct((S, D), f32),
    )(y.astype(jnp.bfloat16), p1c, p2c, g1c, g2c)
    return out.reshape(1, S, D)


# speedup vs baseline: 1.6549x; 1.0046x over previous
import jax
import jax.numpy as jnp
import numpy as np
from jax.experimental import pallas as pl
from jax.experimental.pallas import tpu as pltpu

E = 8
D = 768
H = 12
DH = D // H
DR = D // 2
S = 2048
NA = 2 * S
BA = 128
NB = NA // BA
P_MAX = NB + E - 1
BP = 512
BT = 256


def _bdot(a, b, dims):
    return jax.lax.dot_general(
        a.astype(jnp.bfloat16), b.astype(jnp.bfloat16),
        dimension_numbers=(dims, ((), ())),
        preferred_element_type=jnp.float32)


def _shift_down_lanes(a, k):
    return jnp.concatenate([jnp.zeros((a.shape[0], k), a.dtype), a[:, :-k]],
                           axis=1)


def _shift_down_subl(a, k):
    return jnp.concatenate([jnp.zeros((k, a.shape[1]), a.dtype), a[:-k, :]],
                           axis=0)


def _router_compute(xb_ref, w1_ref, b1_ref, g_ref, be_ref, w2_ref, b2_ref,
                    p1c_ref, p2c_ref, g1c_ref, g2c_ref, off_ref,
                    blk_ref, exp_ref, act_ref, ini_ref):
    h = _bdot(xb_ref[...], w1_ref[...], ((1,), (1,))) + b1_ref[...]
    mu = jnp.mean(h, axis=-1, keepdims=True)
    var = jnp.mean((h - mu) ** 2, axis=-1, keepdims=True)
    h = (h - mu) / jnp.sqrt(var + 1e-5) * g_ref[...] + be_ref[...]
    h = jax.nn.relu(h)
    logits = _bdot(h, w2_ref[...], ((1,), (1,))) + b2_ref[...]
    probs = jax.nn.softmax(logits, axis=-1)
    ei = jax.lax.broadcasted_iota(jnp.int32, probs.shape, 1)
    m1 = jnp.max(probs, axis=1, keepdims=True)
    i1 = jnp.min(jnp.where(probs == m1, ei, E), axis=1, keepdims=True)
    probs2 = jnp.where(ei == i1, -jnp.inf, probs)
    m2 = jnp.max(probs2, axis=1, keepdims=True)
    i2 = jnp.min(jnp.where(probs2 == m2, ei, E), axis=1, keepdims=True)
    denom = m1 + m2
    g1c_ref[...] = m1 / denom
    g2c_ref[...] = m2 / denom

    ind = ((ei == i1) | (ei == i2)).astype(jnp.float32)
    counts = jnp.sum(ind, axis=0, keepdims=True)
    incl = counts
    for k in (1, 2, 4):
        incl = incl + _shift_down_lanes(incl, k)
    off = jnp.concatenate([jnp.zeros((1, 1), jnp.float32), incl], axis=1)
    off_ref[...] = off.astype(jnp.int32)

    itc = jax.lax.broadcasted_iota(jnp.int32, (S, S), 0)
    itr = jax.lax.broadcasted_iota(jnp.int32, (S, S), 1)
    ltri = (itr < itc).astype(jnp.bfloat16)
    rank = jax.lax.dot_general(
        ltri, ind.astype(jnp.bfloat16), (((1,), (0,)), ((), ())),
        preferred_element_type=jnp.float32)
    pos = off[:, :E] + rank
    p1c_ref[...] = jnp.sum(jnp.where(ei == i1, pos, 0.0), axis=1,
                           keepdims=True).astype(jnp.int32)
    p2c_ref[...] = jnp.sum(jnp.where(ei == i2, pos, 0.0), axis=1,
                           keepdims=True).astype(jnp.int32)

    edge_lo = jax.lax.broadcasted_iota(jnp.int32, (NB, 1), 0).astype(
        jnp.float32) * BA
    edge_hi = edge_lo + (BA - 1)
    e_lo = jnp.sum((edge_lo >= off).astype(jnp.float32), axis=1,
                   keepdims=True) - 1.0
    e_hi = jnp.sum((edge_hi >= off).astype(jnp.float32), axis=1,
                   keepdims=True) - 1.0
    n = e_hi - e_lo + 1.0
    cincl = n
    k = 1
    while k < NB:
        cincl = cincl + _shift_down_subl(cincl, k)
        k *= 2
    cexcl = cincl - n
    total = cincl[NB - 1, 0]
    prow = jax.lax.broadcasted_iota(jnp.int32, (1, P_MAX), 1).astype(
        jnp.float32)
    peff = jnp.minimum(prow, total - 1.0)
    in_bp = (peff >= cexcl) & (peff < cincl)
    bcol = jax.lax.broadcasted_iota(jnp.int32, (NB, 1), 0).astype(jnp.float32)
    blk = jnp.sum(jnp.where(in_bp, bcol, 0.0), axis=0, keepdims=True)
    ex = jnp.sum(jnp.where(in_bp, e_lo + peff - cexcl, 0.0), axis=0,
                 keepdims=True)
    act = (prow < total).astype(jnp.int32)
    blk_prev = jnp.concatenate([blk[:, :1] - 1.0, blk[:, :-1]], axis=1)
    ini = (blk != blk_prev).astype(jnp.int32)
    blk_ref[...] = blk.astype(jnp.int32)
    exp_ref[...] = ex.astype(jnp.int32)
    act_ref[...] = act
    ini_ref[...] = ini


def _prep_body(xb_ref, w1_ref, b1_ref, g_ref, be_ref, w2_ref, b2_ref,
               wk_ref, bk_ref, wv_ref, bv_ref,
               p1_ref, p2_ref, g1c_ref, g2c_ref, off_ref,
               blk_ref, exp_ref, act_ref, ini_ref,
               k_out, v_out, km_out, xs_out):
    e = pl.program_id(0)

    @pl.when(e == 0)
    def _():
        _router_compute(xb_ref, w1_ref, b1_ref, g_ref, be_ref, w2_ref,
                        b2_ref, p1_ref, p2_ref, g1c_ref, g2c_ref, off_ref,
                        blk_ref, exp_ref, act_ref, ini_ref)

    k = _bdot(xb_ref[...], wk_ref[0], ((1,), (1,))) + bk_ref[0]
    k_out[0] = k.astype(jnp.bfloat16)
    k2 = k * k
    dcol = jax.lax.broadcasted_iota(jnp.int32, (D, H), 0)
    hrow = jax.lax.broadcasted_iota(jnp.int32, (D, H), 1)
    hm = (dcol // DH == hrow).astype(jnp.float32)
    kn2 = _bdot(k2, hm, ((1,), (0,)))
    km_out[0] = jnp.max(kn2, axis=0, keepdims=True)
    v = _bdot(xb_ref[...], wv_ref[0], ((1,), (1,))) + bv_ref[0]
    vb = v.astype(jnp.bfloat16)
    pat = jnp.concatenate([jnp.ones((S, 1), jnp.bfloat16),
                           jnp.zeros((S, DH - 1), jnp.bfloat16)], axis=1)
    for h in range(H):
        vh = jnp.concatenate([vb[:, h * DH:(h + 1) * DH], pat], axis=1)
        v_out[0, :, 2 * h * DH:(2 * h + 2) * DH] = vh
    prow = e * BP + jax.lax.broadcasted_iota(jnp.int32, (1, BP), 1)
    eqt = ((p1_ref[...] == prow) | (p2_ref[...] == prow)).astype(jnp.bfloat16)
    xs = jax.lax.dot_general(
        eqt, xb_ref[...], (((0,), (0,)), ((), ())),
        preferred_element_type=jnp.float32)
    xs_out[...] = xs.astype(jnp.bfloat16)


def _attn_body(blk_s, exp_s, act_s, ini_s, off_s,
               xs_ref, k_ref, v_ref, km_ref, wq_ref, bq_ref, wo_ref, bo_ref,
               y_ref):
    p = pl.program_id(0)
    e = exp_s[p]
    scale = 1.0 / np.sqrt(DH)
    q = _bdot(xs_ref[...], wq_ref[0], ((1,), (1,))) + bq_ref[0]
    qs = q * scale
    qb = qs.astype(jnp.bfloat16)
    q2 = qs * qs
    km = km_ref[0]
    o_parts = []
    for h in range(H):
        sl = slice(h * DH, (h + 1) * DH)
        scores = jax.lax.dot_general(
            qb[:, sl], k_ref[0][:, sl], (((1,), (1,)), ((), ())),
            preferred_element_type=jnp.float32)
        qn2 = jnp.sum(q2[:, sl], axis=1, keepdims=True)
        b = jnp.sqrt(qn2 * km[0, h]) + 1.0
        ex = jnp.exp(scores - b)
        av = jax.lax.dot_general(
            ex.astype(jnp.bfloat16), v_ref[0][:, h * 2 * DH:(h + 1) * 2 * DH],
            (((1,), (0,)), ((), ())), preferred_element_type=jnp.float32)
        oh = av[:, :DH] / av[:, DH:DH + 1]
        o_parts.append(oh)
    o = jnp.concatenate(o_parts, axis=1)
    o = _bdot(o, wo_ref[0], ((1,), (1,))) + bo_ref[0]
    piota = blk_s[p] * BA + jax.lax.broadcasted_iota(jnp.int32, (BA, 1), 0)
    rowmask = (piota >= off_s[e]) & (piota < off_s[e + 1]) & (act_s[p] > 0)
    contrib = jnp.where(rowmask, o, 0.0)

    @pl.when(ini_s[p] == 1)
    def _():
        y_ref[...] = contrib

    @pl.when(ini_s[p] == 0)
    def _():
        y_ref[...] = y_ref[...] + contrib


def _combine_body(yb_ref, p1_ref, p2_ref, g1_ref, g2_ref, out_ref):
    prow = jax.lax.broadcasted_iota(jnp.int32, (1, NA), 1)
    eq1 = (p1_ref[...] == prow).astype(jnp.bfloat16)
    eq2 = (p2_ref[...] == prow).astype(jnp.bfloat16)
    a1 = jax.lax.dot_general(eq1, yb_ref[...], (((1,), (0,)), ((), ())),
                             preferred_element_type=jnp.float32)
    a2 = jax.lax.dot_general(eq2, yb_ref[...], (((1,), (0,)), ((), ())),
                             preferred_element_type=jnp.float32)
    out_ref[...] = g1_ref[...] * a1 + g2_ref[...] * a2


def kernel(x, Wq, bq, Wk, bk, Wv, bv, Wo, bo, rW1, rb1, ln_g, ln_b, rW2, rb2):
    x2 = x[0]
    xb = x2.astype(jnp.bfloat16)
    i32 = jnp.int32
    f32 = jnp.float32

    b3 = lambda a: a.reshape(E, 1, D)
    wspec = pl.BlockSpec((1, D, D), lambda e: (e, 0, 0))
    bspec = pl.BlockSpec((1, 1, D), lambda e: (e, 0, 0))
    full = lambda shp: pl.BlockSpec(shp, lambda e: tuple(0 for _ in shp))
    outc = full

    (p1c, p2c, g1c, g2c, off, blk, ex, act, ini,
     kall, vall, kmax, xs) = pl.pallas_call(
        _prep_body,
        grid=(E,),
        in_specs=[full((S, D)),
                  full((DR, D)), full((1, DR)), full((1, DR)), full((1, DR)),
                  full((E, DR)), full((1, E)),
                  wspec, bspec, wspec, bspec],
        out_specs=(outc((S, 1)), outc((S, 1)), outc((S, 1)), outc((S, 1)),
                   outc((1, E + 1)), outc((1, P_MAX)), outc((1, P_MAX)),
                   outc((1, P_MAX)), outc((1, P_MAX)),
                   pl.BlockSpec((1, S, D), lambda e: (e, 0, 0)),
                   pl.BlockSpec((1, S, 2 * D), lambda e: (e, 0, 0)),
                   pl.BlockSpec((1, 1, H), lambda e: (e, 0, 0)),
                   pl.BlockSpec((BP, D), lambda e: (e, 0))),
        out_shape=(
            jax.ShapeDtypeStruct((S, 1), i32),
            jax.ShapeDtypeStruct((S, 1), i32),
            jax.ShapeDtypeStruct((S, 1), f32),
            jax.ShapeDtypeStruct((S, 1), f32),
            jax.ShapeDtypeStruct((1, E + 1), i32),
            jax.ShapeDtypeStruct((1, P_MAX), i32),
            jax.ShapeDtypeStruct((1, P_MAX), i32),
            jax.ShapeDtypeStruct((1, P_MAX), i32),
            jax.ShapeDtypeStruct((1, P_MAX), i32),
            jax.ShapeDtypeStruct((E, S, D), jnp.bfloat16),
            jax.ShapeDtypeStruct((E, S, 2 * D), jnp.bfloat16),
            jax.ShapeDtypeStruct((E, 1, H), jnp.float32),
            jax.ShapeDtypeStruct((NA, D), jnp.bfloat16)),
    )(xb, rW1, rb1.reshape(1, DR), ln_g.reshape(1, DR),
      ln_b.reshape(1, DR), rW2, rb2.reshape(1, E),
      Wk, b3(bk), Wv, b3(bv))

    grid_spec = pltpu.PrefetchScalarGridSpec(
        num_scalar_prefetch=5,
        grid=(P_MAX,),
        in_specs=[
            pl.BlockSpec((BA, D), lambda p, b, e, a, i, o: (b[p], 0)),
            pl.BlockSpec((1, S, D), lambda p, b, e, a, i, o: (e[p], 0, 0)),
            pl.BlockSpec((1, S, 2 * D), lambda p, b, e, a, i, o: (e[p], 0, 0)),
            pl.BlockSpec((1, 1, H), lambda p, b, e, a, i, o: (e[p], 0, 0)),
            pl.BlockSpec((1, D, D), lambda p, b, e, a, i, o: (e[p], 0, 0)),
            pl.BlockSpec((1, 1, D), lambda p, b, e, a, i, o: (e[p], 0, 0)),
            pl.BlockSpec((1, D, D), lambda p, b, e, a, i, o: (e[p], 0, 0)),
            pl.BlockSpec((1, 1, D), lambda p, b, e, a, i, o: (e[p], 0, 0)),
        ],
        out_specs=pl.BlockSpec((BA, D), lambda p, b, e, a, i, o: (b[p], 0)),
    )
    y = pl.pallas_call(
        _attn_body,
        grid_spec=grid_spec,
        out_shape=jax.ShapeDtypeStruct((NA, D), f32),
    )(blk.reshape(P_MAX), ex.reshape(P_MAX), act.reshape(P_MAX),
      ini.reshape(P_MAX), off.reshape(E + 1),
      xs, kall, vall, kmax, Wq, b3(bq), Wo, b3(bo))

    out = pl.pallas_call(
        _combine_body,
        grid=(S // BT,),
        in_specs=[pl.BlockSpec((NA, D), lambda j: (0, 0)),
                  pl.BlockSpec((BT, 1), lambda j: (j, 0)),
                  pl.BlockSpec((BT, 1), lambda j: (j, 0)),
                  pl.BlockSpec((BT, 1), lambda j: (j, 0)),
                  pl.BlockSpec((BT, 1), lambda j: (j, 0))],
        out_specs=pl.BlockSpec((BT, D), lambda j: (j, 0)),
        out_shape=jax.ShapeDtypeStruct((S, D), f32),
    )(y.astype(jnp.bfloat16), p1c, p2c, g1c, g2c)
    return out.reshape(1, S, D)
